# Initial kernel scaffold; baseline (speedup 1.0000x reference)
import jax
import jax.numpy as jnp
from jax.experimental import pallas as pl


def kernel(static_table, time_table, pos_table, W_ih, W_hh, b_ih, b_hh, W_trans, b_trans, cas_times, cas_history, lengths):
    raise NotImplementedError("write your pallas kernel here")


# speedup vs baseline: 8.0183x; 8.0183x over previous
import jax
import jax.numpy as jnp
from jax.experimental import pallas as pl
from jax.experimental.pallas import tpu as pltpu
from jax.experimental.pallas import tpu_sc as plsc

B = 4096
L = 50
D = 128
TIME_NUM = 50
TIME_PAD = 64
MAX_TIME = 1000.0
GATHER_WINDOW = 128


def _sc_gather(table, flat_idx):
    n = flat_idx.shape[0]
    idx2d = flat_idx.reshape(1, n)
    mesh = plsc.VectorSubcoreMesh(core_axis_name="core", subcore_axis_name="subcore")

    @pl.kernel(
        out_type=jax.ShapeDtypeStruct((n, table.shape[1]), table.dtype),
        mesh=mesh,
    )
    def kern(x_hbm, i_hbm, o_hbm):
        def body(i_vmem, o_vmem):
            pltpu.sync_copy(x_hbm.at[i_vmem.at[0]], o_vmem)

        pltpu.emit_pipeline(
            body,
            grid=(n // GATHER_WINDOW,),
            in_specs=[pl.BlockSpec((1, GATHER_WINDOW), index_map=lambda i: (0, i))],
            out_specs=[
                pl.BlockSpec((GATHER_WINDOW, table.shape[1]), index_map=lambda i: (i, 0))
            ],
            core_axis_name=("core", "subcore"),
            dimension_semantics=(pltpu.PARALLEL,),
        )(i_hbm, o_hbm)

    return kern(table, idx2d)


def _lstm_kernel(x_ref, tidx_ref, len_ref, pos_ref, time_ref, wih_ref, whh_ref,
                 bias_ref, wtr_ref, btr_ref, out_ref, h_ref, c_ref):
    t = pl.program_id(0)

    @pl.when(t == 0)
    def _():
        h_ref[...] = jnp.zeros_like(h_ref)
        c_ref[...] = jnp.zeros_like(c_ref)

    xt = x_ref[0]
    tcol = tidx_ref[0]
    onehot = (tcol == jax.lax.broadcasted_iota(
        jnp.int32, (B, TIME_PAD), 1)).astype(jnp.float32)
    xt = xt + jnp.dot(onehot, time_ref[...], preferred_element_type=jnp.float32)
    xt = xt + pos_ref[0]

    h = h_ref[...]
    c = c_ref[...]
    gates = (jnp.dot(xt, wih_ref[...], preferred_element_type=jnp.float32)
             + jnp.dot(h, whh_ref[...], preferred_element_type=jnp.float32)
             + bias_ref[...])
    gi = jax.nn.sigmoid(gates[:, 0:D])
    gf = jax.nn.sigmoid(gates[:, D:2 * D])
    gg = jnp.tanh(gates[:, 2 * D:3 * D])
    go = jax.nn.sigmoid(gates[:, 3 * D:4 * D])
    c_new = gf * c + gi * gg
    h_new = go * jnp.tanh(c_new)
    mask = t < len_ref[...]
    h = jnp.where(mask, h_new, h)
    h_ref[...] = h
    c_ref[...] = jnp.where(mask, c_new, c)

    @pl.when(t == L - 1)
    def _():
        out_ref[...] = jax.nn.relu(
            jnp.dot(h, wtr_ref[...], preferred_element_type=jnp.float32)
            + btr_ref[...])


def _run_lstm(x_lbd, tidx_t, lengths, pos_slice, time_pad, wih_t, whh_t, bias,
              wtr_t, btr):
    return pl.pallas_call(
        _lstm_kernel,
        grid=(L,),
        in_specs=[
            pl.BlockSpec((1, B, D), lambda t: (t, 0, 0)),
            pl.BlockSpec((1, B, 1), lambda t: (t, 0, 0)),
            pl.BlockSpec((B, 1), lambda t: (0, 0)),
            pl.BlockSpec((1, 1, D), lambda t: (t, 0, 0)),
            pl.BlockSpec((TIME_PAD, D), lambda t: (0, 0)),
            pl.BlockSpec((D, 4 * D), lambda t: (0, 0)),
            pl.BlockSpec((D, 4 * D), lambda t: (0, 0)),
            pl.BlockSpec((1, 4 * D), lambda t: (0, 0)),
            pl.BlockSpec((D, D), lambda t: (0, 0)),
            pl.BlockSpec((1, D), lambda t: (0, 0)),
        ],
        out_specs=pl.BlockSpec((B, D), lambda t: (0, 0)),
        out_shape=jax.ShapeDtypeStruct((B, D), jnp.float32),
        scratch_shapes=[
            pltpu.VMEM((B, D), jnp.float32),
            pltpu.VMEM((B, D), jnp.float32),
        ],
        compiler_params=pltpu.CompilerParams(
            dimension_semantics=("arbitrary",)),
    )(x_lbd, tidx_t, lengths, pos_slice, time_pad, wih_t, whh_t, bias, wtr_t, btr)


def kernel(static_table, time_table, pos_table, W_ih, W_hh, b_ih, b_hh,
           W_trans, b_trans, cas_times, cas_history, lengths):
    tidx = jnp.clip(
        jnp.floor(cas_times / MAX_TIME * TIME_NUM).astype(jnp.int32),
        0, TIME_NUM - 1)
    tidx_t = tidx.T.reshape(L, B, 1)
    idx_flat = cas_history.T.reshape(L * B)
    x_lbd = _sc_gather(static_table, idx_flat).reshape(L, B, D)

    pos_slice = pos_table[:L].reshape(L, 1, D)
    time_pad = jnp.zeros((TIME_PAD, D), jnp.float32).at[:TIME_NUM].set(time_table)
    bias = (b_ih + b_hh).reshape(1, 4 * D)
    out = _run_lstm(x_lbd, tidx_t, lengths.reshape(B, 1), pos_slice, time_pad,
                    W_ih.T, W_hh.T, bias, W_trans.T, b_trans.reshape(1, D))
    return out
